# Initial kernel scaffold; baseline (speedup 1.0000x reference)
import functools

import jax
import jax.numpy as jnp
from jax.experimental import pallas as pl
from jax.experimental.pallas import tpu as pltpu

B = 32
P = 8732
NT = 12
R, L = 69, 128
PP = R * L
NC = 2


def _body(t_ref, pri_ref, loc_ref, conf_ref, fc_ref, out_ref):
    b = pl.program_id(0)
    lin = (jax.lax.broadcasted_iota(jnp.int32, (R, L), 0) * L
           + jax.lax.broadcasted_iota(jnp.int32, (R, L), 1))
    valid = lin < P

    pcx = pri_ref[0]
    pcy = pri_ref[1]
    pw = pri_ref[2]
    ph = pri_ref[3]
    px0 = pcx - pw / 2.0
    py0 = pcy - ph / 2.0
    px1 = pcx + pw / 2.0
    py1 = pcy + ph / 2.0
    parea = (px1 - px0) * (py1 - py0)

    bto = jnp.full((R, L), -1.0, jnp.float32)
    bti = jnp.zeros((R, L), jnp.int32)
    bps = []
    for j in range(NT):
        tx0 = t_ref[0, j * 13 + 0]
        ty0 = t_ref[0, j * 13 + 1]
        tx1 = t_ref[0, j * 13 + 2]
        ty1 = t_ref[0, j * 13 + 3]
        tarea = (tx1 - tx0) * (ty1 - ty0)
        iw = jnp.maximum(jnp.minimum(tx1, px1) - jnp.maximum(tx0, px0), 0.0)
        ih = jnp.maximum(jnp.minimum(ty1, py1) - jnp.maximum(ty0, py0), 0.0)
        inter = iw * ih
        iou = inter / (tarea + parea - inter)
        iou = jnp.where(valid, iou, -1.0)
        upd = iou > bto
        bto = jnp.where(upd, iou, bto)
        bti = jnp.where(upd, j, bti)
        m = jnp.max(iou)
        bp = jnp.min(jnp.where(iou == m, lin, jnp.int32(2 ** 30)))
        bps.append(bp)
    for j in range(NT):
        mask = lin == bps[j]
        bto = jnp.where(mask, 2.0, bto)
        bti = jnp.where(mask, j, bti)
    pos = bto >= 0.5

    mc = []
    for c in range(12):
        acc = jnp.full((R, L), t_ref[0, c], jnp.float32)
        for j in range(1, NT):
            acc = jnp.where(bti == j, t_ref[0, j * 13 + c], acc)
        mc.append(acc)

    vw = 0.1 * pw
    vh = 0.1 * ph
    g = [((mc[0] + mc[2]) / 2.0 - pcx) / vw,
         ((mc[1] + mc[3]) / 2.0 - pcy) / vh,
         jnp.log((mc[2] - mc[0]) / pw) / 0.2,
         jnp.log((mc[3] - mc[1]) / ph) / 0.2]
    ll = jnp.float32(0.0)
    for k in range(4):
        d = loc_ref[0, k] - g[k]
        ad = jnp.abs(d)
        s = jnp.where(ad < 1.0, 0.5 * d * d, ad - 0.5)
        ll = ll + jnp.sum(jnp.where(pos, s, 0.0))
    lfc = jnp.float32(0.0)
    for k in range(8):
        ctr = pcx if k % 2 == 0 else pcy
        wh = vw if k % 2 == 0 else vh
        gk = (mc[4 + k] - ctr) / wh
        d = fc_ref[0, k] - gk
        ad = jnp.abs(d)
        s = jnp.where(ad < 1.0, 0.5 * d * d, ad - 0.5)
        lfc = lfc + jnp.sum(jnp.where(pos, s, 0.0))

    c0 = conf_ref[0, 0]
    c1 = conf_ref[0, 1]
    mx = jnp.maximum(c0, c1)
    lse = jnp.log(jnp.exp(c0 - mx) + jnp.exp(c1 - mx)) + mx
    ce_pos_sum = jnp.sum(jnp.where(pos, lse - c1, 0.0))
    v = jnp.where(pos | jnp.logical_not(valid), 0.0, lse - c0)

    np_f = jnp.sum(jnp.where(pos, 1.0, 0.0))
    num_neg = jnp.minimum(3.0 * np_f, jnp.float32(P - 1))
    kk = jnp.minimum(num_neg, jnp.float32(P) - np_f)

    def bs_body(i, lo):
        cand = lo | (jnp.int32(1) << (30 - i))
        t = jax.lax.bitcast_convert_type(cand, jnp.float32)
        cnt = jnp.sum(jnp.where(v > t, 1.0, 0.0))
        return jnp.where(cnt >= kk, cand, lo)

    lo = jax.lax.fori_loop(0, 31, bs_body, jnp.int32(0))
    thr = jax.lax.bitcast_convert_type(lo + 1, jnp.float32)
    gt = v > thr
    cnt_gt = jnp.sum(jnp.where(gt, 1.0, 0.0))
    sum_gt = jnp.sum(jnp.where(gt, v, 0.0))
    topk = sum_gt + (kk - cnt_gt) * thr
    topk = jnp.where(kk > 0.0, topk, 0.0)
    lc = ce_pos_sum + topk

    lane = jax.lax.broadcasted_iota(jnp.int32, (1, L), 1)
    vec = (jnp.where(lane == 0, ll, 0.0)
           + jnp.where(lane == 1, lc, 0.0)
           + jnp.where(lane == 2, lfc, 0.0)
           + jnp.where(lane == 3, np_f, 0.0))

    @pl.when(b == 0)
    def _():
        out_ref[...] = vec

    @pl.when(b > 0)
    def _():
        out_ref[...] = out_ref[...] + vec

    @pl.when(b == B - 1)
    def _():
        tot = out_ref[...]
        n = jnp.sum(jnp.where(lane == 3, tot, 0.0))
        out_ref[...] = tot / n


@jax.jit
def _run(t, pri, loc, conf, fc):
    return pl.pallas_call(
        _body,
        grid=(B,),
        in_specs=[
            pl.BlockSpec((1, 160), lambda b: (b, 0), memory_space=pltpu.SMEM),
            pl.BlockSpec((4, R, L), lambda b: (0, 0, 0)),
            pl.BlockSpec((1, 4, R, L), lambda b: (b, 0, 0, 0)),
            pl.BlockSpec((1, NC, R, L), lambda b: (b, 0, 0, 0)),
            pl.BlockSpec((1, 8, R, L), lambda b: (b, 0, 0, 0)),
        ],
        out_specs=pl.BlockSpec((1, L), lambda b: (0, 0)),
        out_shape=jax.ShapeDtypeStruct((1, L), jnp.float32),
    )(t, pri, loc, conf, fc)


def kernel(loc_data, conf_data, priors, four_corners_data, targets):
    pad = PP - P

    def prep(x):
        x = jnp.pad(x, ((0, 0), (0, pad), (0, 0)))
        return x.transpose(0, 2, 1).reshape(B, -1, R, L)

    loc = prep(loc_data)
    conf = prep(conf_data)
    fc = prep(four_corners_data)
    pri_pad = jnp.tile(jnp.array([[-10.0, -10.0, 1.0, 1.0]], jnp.float32),
                       (pad, 1))
    pri = (jnp.concatenate([priors, pri_pad], axis=0)
           .transpose(1, 0).reshape(4, R, L))
    t = jnp.pad(targets.reshape(B, NT * 13), ((0, 0), (0, 160 - NT * 13)))

    out = _run(t, pri, loc, conf, fc)
    return (out[0, 0], out[0, 1], out[0, 2])


# speedup vs baseline: 15.1704x; 15.1704x over previous
import functools

import jax
import jax.numpy as jnp
from jax.experimental import pallas as pl
from jax.experimental.pallas import tpu as pltpu

B = 32
P = 8732
NT = 12
R, L = 69, 128
PP = R * L
NC = 2


def _body(t_ref, pri_ref, loc_ref, conf_ref, fc_ref, out_ref):
    b = pl.program_id(0)
    lin = (jax.lax.broadcasted_iota(jnp.int32, (R, L), 0) * L
           + jax.lax.broadcasted_iota(jnp.int32, (R, L), 1))
    valid = lin < P

    pcx = pri_ref[0]
    pcy = pri_ref[1]
    pw = pri_ref[2]
    ph = pri_ref[3]
    px0 = pcx - pw / 2.0
    py0 = pcy - ph / 2.0
    px1 = pcx + pw / 2.0
    py1 = pcy + ph / 2.0
    parea = (px1 - px0) * (py1 - py0)

    bto = jnp.full((R, L), -1.0, jnp.float32)
    bti = jnp.zeros((R, L), jnp.int32)
    bps = []
    for j in range(NT):
        tx0 = t_ref[0, 0, j * 13 + 0]
        ty0 = t_ref[0, 0, j * 13 + 1]
        tx1 = t_ref[0, 0, j * 13 + 2]
        ty1 = t_ref[0, 0, j * 13 + 3]
        tarea = (tx1 - tx0) * (ty1 - ty0)
        iw = jnp.maximum(jnp.minimum(tx1, px1) - jnp.maximum(tx0, px0), 0.0)
        ih = jnp.maximum(jnp.minimum(ty1, py1) - jnp.maximum(ty0, py0), 0.0)
        inter = iw * ih
        iou = inter / (tarea + parea - inter)
        iou = jnp.where(valid, iou, -1.0)
        upd = iou > bto
        bto = jnp.where(upd, iou, bto)
        bti = jnp.where(upd, j, bti)
        m = jnp.max(iou)
        bp = jnp.min(jnp.where(iou == m, lin, jnp.int32(2 ** 30)))
        bps.append(bp)
    for j in range(NT):
        mask = lin == bps[j]
        bto = jnp.where(mask, 2.0, bto)
        bti = jnp.where(mask, j, bti)
    pos = bto >= 0.5

    mc = []
    for c in range(12):
        acc = jnp.full((R, L), t_ref[0, 0, c], jnp.float32)
        for j in range(1, NT):
            acc = jnp.where(bti == j, t_ref[0, 0, j * 13 + c], acc)
        mc.append(acc)

    vw = 0.1 * pw
    vh = 0.1 * ph
    g = [((mc[0] + mc[2]) / 2.0 - pcx) / vw,
         ((mc[1] + mc[3]) / 2.0 - pcy) / vh,
         jnp.log((mc[2] - mc[0]) / pw) / 0.2,
         jnp.log((mc[3] - mc[1]) / ph) / 0.2]
    ll = jnp.float32(0.0)
    for k in range(4):
        d = loc_ref[0, k] - g[k]
        ad = jnp.abs(d)
        s = jnp.where(ad < 1.0, 0.5 * d * d, ad - 0.5)
        ll = ll + jnp.sum(jnp.where(pos, s, 0.0))
    lfc = jnp.float32(0.0)
    for k in range(8):
        ctr = pcx if k % 2 == 0 else pcy
        wh = vw if k % 2 == 0 else vh
        gk = (mc[4 + k] - ctr) / wh
        d = fc_ref[0, k] - gk
        ad = jnp.abs(d)
        s = jnp.where(ad < 1.0, 0.5 * d * d, ad - 0.5)
        lfc = lfc + jnp.sum(jnp.where(pos, s, 0.0))

    c0 = conf_ref[0, 0]
    c1 = conf_ref[0, 1]
    mx = jnp.maximum(c0, c1)
    lse = jnp.log(jnp.exp(c0 - mx) + jnp.exp(c1 - mx)) + mx
    ce_pos_sum = jnp.sum(jnp.where(pos, lse - c1, 0.0))
    v = jnp.where(pos | jnp.logical_not(valid), 0.0, lse - c0)

    np_f = jnp.sum(jnp.where(pos, 1.0, 0.0))
    num_neg = jnp.minimum(3.0 * np_f, jnp.float32(P - 1))
    kk = jnp.minimum(num_neg, jnp.float32(P) - np_f)

    def bs_body(i, lo):
        cand = lo | (jnp.int32(1) << (30 - i))
        t = jax.lax.bitcast_convert_type(cand, jnp.float32)
        cnt = jnp.sum(jnp.where(v > t, 1.0, 0.0))
        return jnp.where(cnt >= kk, cand, lo)

    lo = jax.lax.fori_loop(0, 31, bs_body, jnp.int32(0))
    thr = jax.lax.bitcast_convert_type(lo + 1, jnp.float32)
    gt = v > thr
    cnt_gt = jnp.sum(jnp.where(gt, 1.0, 0.0))
    sum_gt = jnp.sum(jnp.where(gt, v, 0.0))
    topk = sum_gt + (kk - cnt_gt) * thr
    topk = jnp.where(kk > 0.0, topk, 0.0)
    lc = ce_pos_sum + topk

    lane = jax.lax.broadcasted_iota(jnp.int32, (1, L), 1)
    vec = (jnp.where(lane == 0, ll, 0.0)
           + jnp.where(lane == 1, lc, 0.0)
           + jnp.where(lane == 2, lfc, 0.0)
           + jnp.where(lane == 3, np_f, 0.0))

    @pl.when(b == 0)
    def _():
        out_ref[...] = vec

    @pl.when(b > 0)
    def _():
        out_ref[...] = out_ref[...] + vec

    @pl.when(b == B - 1)
    def _():
        tot = out_ref[...]
        n = jnp.sum(jnp.where(lane == 3, tot, 0.0))
        out_ref[...] = tot / n


@jax.jit
def _run(t, pri, loc, conf, fc):
    return pl.pallas_call(
        _body,
        grid=(B,),
        in_specs=[
            pl.BlockSpec((1, 1, 160), lambda b: (b, 0, 0),
                         memory_space=pltpu.SMEM),
            pl.BlockSpec((4, R, L), lambda b: (0, 0, 0)),
            pl.BlockSpec((1, 4, R, L), lambda b: (b, 0, 0, 0)),
            pl.BlockSpec((1, NC, R, L), lambda b: (b, 0, 0, 0)),
            pl.BlockSpec((1, 8, R, L), lambda b: (b, 0, 0, 0)),
        ],
        out_specs=pl.BlockSpec((1, L), lambda b: (0, 0)),
        out_shape=jax.ShapeDtypeStruct((1, L), jnp.float32),
    )(t, pri, loc, conf, fc)


def kernel(loc_data, conf_data, priors, four_corners_data, targets):
    pad = PP - P

    def prep(x):
        x = jnp.pad(x, ((0, 0), (0, pad), (0, 0)))
        return x.transpose(0, 2, 1).reshape(B, -1, R, L)

    loc = prep(loc_data)
    conf = prep(conf_data)
    fc = prep(four_corners_data)
    pri_pad = jnp.tile(jnp.array([[-10.0, -10.0, 1.0, 1.0]], jnp.float32),
                       (pad, 1))
    pri = (jnp.concatenate([priors, pri_pad], axis=0)
           .transpose(1, 0).reshape(4, R, L))
    t = jnp.pad(targets.reshape(B, 1, NT * 13),
                ((0, 0), (0, 0), (0, 160 - NT * 13)))

    out = _run(t, pri, loc, conf, fc)
    return (out[0, 0], out[0, 1], out[0, 2])


# speedup vs baseline: 27.4481x; 1.8093x over previous
import functools

import jax
import jax.numpy as jnp
from jax.experimental import pallas as pl
from jax.experimental.pallas import tpu as pltpu

B = 32
P = 8732
NT = 12
R, L = 69, 128
PP = R * L
NC = 2


def _phase1(t_ref, pri_ref, loc_ref, conf_ref, fc_ref, meta_ref, v_ref):
    lin = (jax.lax.broadcasted_iota(jnp.int32, (R, L), 0) * L
           + jax.lax.broadcasted_iota(jnp.int32, (R, L), 1))
    valid = lin < P

    pcx = pri_ref[0]
    pcy = pri_ref[1]
    pw = pri_ref[2]
    ph = pri_ref[3]
    px0 = pcx - pw / 2.0
    py0 = pcy - ph / 2.0
    px1 = pcx + pw / 2.0
    py1 = pcy + ph / 2.0
    parea = (px1 - px0) * (py1 - py0)

    bto = jnp.full((R, L), -1.0, jnp.float32)
    bti = jnp.zeros((R, L), jnp.int32)
    bps = []
    for j in range(NT):
        tx0 = t_ref[0, 0, j * 13 + 0]
        ty0 = t_ref[0, 0, j * 13 + 1]
        tx1 = t_ref[0, 0, j * 13 + 2]
        ty1 = t_ref[0, 0, j * 13 + 3]
        tarea = (tx1 - tx0) * (ty1 - ty0)
        iw = jnp.maximum(jnp.minimum(tx1, px1) - jnp.maximum(tx0, px0), 0.0)
        ih = jnp.maximum(jnp.minimum(ty1, py1) - jnp.maximum(ty0, py0), 0.0)
        inter = iw * ih
        iou = inter / (tarea + parea - inter)
        iou = jnp.where(valid, iou, -1.0)
        upd = iou > bto
        bto = jnp.where(upd, iou, bto)
        bti = jnp.where(upd, j, bti)
        m = jnp.max(iou)
        bp = jnp.min(jnp.where(iou == m, lin, jnp.int32(2 ** 30)))
        bps.append(bp)
    for j in range(NT):
        mask = lin == bps[j]
        bto = jnp.where(mask, 2.0, bto)
        bti = jnp.where(mask, j, bti)
    pos = bto >= 0.5

    mc = []
    for c in range(12):
        acc = jnp.full((R, L), t_ref[0, 0, c], jnp.float32)
        for j in range(1, NT):
            acc = jnp.where(bti == j, t_ref[0, 0, j * 13 + c], acc)
        mc.append(acc)

    vw = 0.1 * pw
    vh = 0.1 * ph
    g = [((mc[0] + mc[2]) / 2.0 - pcx) / vw,
         ((mc[1] + mc[3]) / 2.0 - pcy) / vh,
         jnp.log((mc[2] - mc[0]) / pw) / 0.2,
         jnp.log((mc[3] - mc[1]) / ph) / 0.2]
    ll_p = jnp.zeros((R, L), jnp.float32)
    for k in range(4):
        d = loc_ref[0, k] - g[k]
        ad = jnp.abs(d)
        ll_p = ll_p + jnp.where(ad < 1.0, 0.5 * d * d, ad - 0.5)
    ll = jnp.sum(jnp.where(pos, ll_p, 0.0))
    lfc_p = jnp.zeros((R, L), jnp.float32)
    for k in range(8):
        ctr = pcx if k % 2 == 0 else pcy
        wh = vw if k % 2 == 0 else vh
        gk = (mc[4 + k] - ctr) / wh
        d = fc_ref[0, k] - gk
        ad = jnp.abs(d)
        lfc_p = lfc_p + jnp.where(ad < 1.0, 0.5 * d * d, ad - 0.5)
    lfc = jnp.sum(jnp.where(pos, lfc_p, 0.0))

    c0 = conf_ref[0, 0]
    c1 = conf_ref[0, 1]
    mx = jnp.maximum(c0, c1)
    lse = jnp.log(jnp.exp(c0 - mx) + jnp.exp(c1 - mx)) + mx
    ce_pos_sum = jnp.sum(jnp.where(pos, lse - c1, 0.0))
    v_ref[0] = jnp.where(pos | jnp.logical_not(valid), 0.0, lse - c0)

    np_f = jnp.sum(jnp.where(pos, 1.0, 0.0))
    num_neg = jnp.minimum(3.0 * np_f, jnp.float32(P - 1))
    kk = jnp.minimum(num_neg, jnp.float32(P) - np_f)

    lane = jax.lax.broadcasted_iota(jnp.int32, (1, L), 1)
    meta_ref[0] = (jnp.where(lane == 0, ll, 0.0)
                   + jnp.where(lane == 1, ce_pos_sum, 0.0)
                   + jnp.where(lane == 2, lfc, 0.0)
                   + jnp.where(lane == 3, np_f, 0.0)
                   + jnp.where(lane == 4, kk, 0.0))


def _phase2(meta_ref, v_ref, out_ref):
    v = v_ref[...]
    m = meta_ref[...]
    lane = jax.lax.broadcasted_iota(jnp.int32, (B, 1, L), 2)
    kk = jnp.sum(jnp.where(lane == 4, m, 0.0), axis=2, keepdims=True)

    def bs_body(i, lo):
        cand = lo | (jnp.int32(1) << (30 - i))
        t = jax.lax.bitcast_convert_type(cand, jnp.float32)
        cnt = jnp.sum(jnp.sum(jnp.where(v > t, 1.0, 0.0), axis=2,
                              keepdims=True), axis=1, keepdims=True)
        return jnp.where(cnt >= kk, cand, lo)

    lo = jax.lax.fori_loop(0, 31, bs_body, jnp.zeros((B, 1, 1), jnp.int32))
    thr = jax.lax.bitcast_convert_type(lo + 1, jnp.float32)
    gt = v > thr
    cnt_gt = jnp.sum(jnp.sum(jnp.where(gt, 1.0, 0.0), axis=2, keepdims=True),
                     axis=1, keepdims=True)
    sum_gt = jnp.sum(jnp.sum(jnp.where(gt, v, 0.0), axis=2, keepdims=True),
                     axis=1, keepdims=True)
    topk = sum_gt + (kk - cnt_gt) * thr
    topk = jnp.where(kk > 0.0, topk, 0.0)

    ll = jnp.sum(jnp.where(lane == 0, m, 0.0))
    cepos = jnp.sum(jnp.where(lane == 1, m, 0.0))
    lfc = jnp.sum(jnp.where(lane == 2, m, 0.0))
    n = jnp.sum(jnp.where(lane == 3, m, 0.0))
    lc = cepos + jnp.sum(topk)

    olane = jax.lax.broadcasted_iota(jnp.int32, (1, L), 1)
    out_ref[...] = (jnp.where(olane == 0, ll / n, 0.0)
                    + jnp.where(olane == 1, lc / n, 0.0)
                    + jnp.where(olane == 2, lfc / n, 0.0))


@jax.jit
def _run(t, pri, loc, conf, fc):
    meta, v = pl.pallas_call(
        _phase1,
        grid=(B,),
        in_specs=[
            pl.BlockSpec((1, 1, 160), lambda b: (b, 0, 0),
                         memory_space=pltpu.SMEM),
            pl.BlockSpec((4, R, L), lambda b: (0, 0, 0)),
            pl.BlockSpec((1, 4, R, L), lambda b: (b, 0, 0, 0)),
            pl.BlockSpec((1, NC, R, L), lambda b: (b, 0, 0, 0)),
            pl.BlockSpec((1, 8, R, L), lambda b: (b, 0, 0, 0)),
        ],
        out_specs=[
            pl.BlockSpec((1, 1, L), lambda b: (b, 0, 0)),
            pl.BlockSpec((1, R, L), lambda b: (b, 0, 0)),
        ],
        out_shape=[
            jax.ShapeDtypeStruct((B, 1, L), jnp.float32),
            jax.ShapeDtypeStruct((B, R, L), jnp.float32),
        ],
    )(t, pri, loc, conf, fc)
    out = pl.pallas_call(
        _phase2,
        out_shape=jax.ShapeDtypeStruct((1, L), jnp.float32),
    )(meta, v)
    return out


def kernel(loc_data, conf_data, priors, four_corners_data, targets):
    pad = PP - P

    def prep(x):
        x = jnp.pad(x, ((0, 0), (0, pad), (0, 0)))
        return x.transpose(0, 2, 1).reshape(B, -1, R, L)

    loc = prep(loc_data)
    conf = prep(conf_data)
    fc = prep(four_corners_data)
    pri_pad = jnp.tile(jnp.array([[-10.0, -10.0, 1.0, 1.0]], jnp.float32),
                       (pad, 1))
    pri = (jnp.concatenate([priors, pri_pad], axis=0)
           .transpose(1, 0).reshape(4, R, L))
    t = jnp.pad(targets.reshape(B, 1, NT * 13),
                ((0, 0), (0, 0), (0, 160 - NT * 13)))

    out = _run(t, pri, loc, conf, fc)
    return (out[0, 0], out[0, 1], out[0, 2])
